# baseline (speedup 1.0000x reference)
import functools

import numpy as np
import jax
import jax.numpy as jnp
from jax import lax
from jax.experimental import pallas as pl
from jax.experimental.pallas import tpu as pltpu
from jax.experimental.pallas import tpu_sc as plsc

_EPS = float(np.finfo(np.float64).eps)

_B = 2048
_D = 1024
_H = 4096
_E = 8
_NE = 7
_TM = 128
_CAP = 4992
_NT = _CAP // _TM
_XROWS = _CAP + _TM
_NC, _NS, _L = 2, 16, 16
_NW = _NC * _NS
_CHUNK = (2 * _B) // _NW
_TOKW = _B // _NW
_TN1 = 512
_TN2 = 512

@functools.cache
def _sc_mesh():
    return plsc.VectorSubcoreMesh(core_axis_name="c", subcore_axis_name="s",
                                  num_cores=_NC, num_subcores=_NS)


def _gate_body(x_ref, wg_ref, eids_ref, gm_ref, gid_ref):
    xf = x_ref[...]
    logits = jnp.dot(xf, wg_ref[...], preferred_element_type=jnp.float32)
    eidx = lax.broadcasted_iota(jnp.int32, logits.shape, 1)
    l1 = jnp.max(logits, axis=-1, keepdims=True)
    i1 = jnp.min(jnp.where(logits == l1, eidx, _E), axis=-1, keepdims=True)
    m1 = eidx == i1
    logits2 = jnp.where(m1, -jnp.inf, logits)
    l2 = jnp.max(logits2, axis=-1, keepdims=True)
    i2 = jnp.min(jnp.where(logits2 == l2, eidx, _E), axis=-1, keepdims=True)
    t = jnp.exp(l2 - l1)
    g1 = 1.0 / (1.0 + t)
    g2 = t / (1.0 + t)
    eids_ref[...] = jnp.concatenate([i1, i2], axis=1)
    gm_ref[...] = jnp.concatenate(
        [jnp.where(i1 == _NE, 0.0, g1), jnp.where(i2 == _NE, 0.0, g2)], axis=1)
    gid_ref[...] = jnp.where(i1 == _NE, g1, 0.0) + jnp.where(i2 == _NE, g2, 0.0)


def _gate(x, w_gate):
    return pl.pallas_call(
        _gate_body,
        out_shape=(
            jax.ShapeDtypeStruct((_B, 2), jnp.int32),
            jax.ShapeDtypeStruct((_B, 2), jnp.float32),
            jax.ShapeDtypeStruct((_B, 1), jnp.float32),
        ),
    )(x, w_gate)


@functools.cache
def _dispatch_kernel():
    return pl.kernel(
        _dispatch_body,
        out_type=(
            jax.ShapeDtypeStruct((_XROWS, _D), jnp.float32),
            jax.ShapeDtypeStruct((2 * _B,), jnp.int32),
            jax.ShapeDtypeStruct((48,), jnp.int32),
        ),
        mesh=_sc_mesh(),
        compiler_params=pltpu.CompilerParams(needs_layout_passes=False),
        scratch_types=[
            pltpu.VMEM((2 * _B,), jnp.int32),
            pltpu.VMEM((_CHUNK,), jnp.int32),
            pltpu.VMEM((64,), jnp.int32),
            pltpu.VMEM((64,), jnp.int32),
            pltpu.VMEM((64, _D), jnp.float32),
            pltpu.VMEM((48,), jnp.int32),
            pltpu.SemaphoreType.DMA,
        ],
    )


def _dispatch_body(eflat, x, xs, pos, et, ev, posv, ida, idb, xrow, etv, sem):
    w = lax.axis_index("c") * _NS + lax.axis_index("s")
    pltpu.sync_copy(eflat, ev)
    lanes = lax.iota(jnp.int32, _L)

    def scan_body(i, carry):
        ev_v = ev[pl.ds(i * _L, _L)]
        take = (i // (_CHUNK // _L) < w).astype(jnp.int32)
        out = []
        for ex in range(_NE):
            eq = 1 - jnp.minimum(jnp.abs(ev_v - ex), 1)
            out.append(carry[ex] + eq)
            out.append(carry[_NE + ex] + eq * take)
        return tuple(out[0::2]) + tuple(out[1::2])

    zero = jnp.zeros((_L,), jnp.int32)
    acc = lax.fori_loop(0, (2 * _B) // _L, scan_body, (zero,) * (2 * _NE))

    tot = jnp.zeros((_L,), jnp.int32)
    base = jnp.zeros((_L,), jnp.int32)
    for ex in range(_NE):
        sel = 1 - jnp.minimum(jnp.abs(lanes - ex), 1)
        tot = tot + sel * plsc.cumsum(acc[ex])[_L - 1]
        base = base + sel * plsc.cumsum(acc[_NE + ex])[_L - 1]

    is_mlp_lane = 1 - jnp.minimum(jnp.maximum(lanes - (_NE - 1), 0), 1)
    padded = is_mlp_lane * (((tot + (_TM - 1)) >> 7) << 7)
    cum = plsc.cumsum(padded)
    start = cum - padded
    mybase = start + base

    pbase = w * _CHUNK
    runs = [jnp.int32(0)] * _NE
    for v in range(_CHUNK // _L):
        ev_v = ev[pl.ds(pbase + v * _L, _L)]
        slot = jnp.zeros((_L,), jnp.int32)
        for ex in range(_NE):
            eq = 1 - jnp.minimum(jnp.abs(ev_v - ex), 1)
            rank = plsc.cumsum(eq)
            bs = mybase[ex] + runs[ex]
            slot = slot + eq * (bs + rank - 1)
            runs[ex] = runs[ex] + rank[_L - 1]
        is_mlp = 1 - jnp.minimum(jnp.maximum(ev_v - (_NE - 1), 0), 1)
        posv[pl.ds(v * _L, _L)] = is_mlp * slot
        sl_sc = is_mlp * slot + (1 - is_mlp) * _CAP
        if v < 4:
            ida[pl.ds(v * _L, _L)] = sl_sc
        else:
            idb[pl.ds((v - 4) * _L, _L)] = sl_sc

    pltpu.sync_copy(posv, pos.at[pl.ds(pbase, _CHUNK)])

    brow = (w % _NS) * _CHUNK
    for half in range(2):
        pltpu.sync_copy(x.at[pl.ds(brow + half * 64, 64)], xrow)
        idx = ida if half == 0 else idb
        pltpu.async_copy(xrow, xs.at[idx], sem).wait()

    @pl.when(w == 0)
    def _et():
        for vi in range(48 // _L):
            tt = (lax.iota(jnp.int32, _L) + vi * _L) * _TM
            acc = jnp.full((_L,), -1, jnp.int32)
            for ex in range(_NE):
                ge = 1 - jnp.minimum(jnp.maximum(start[ex] - tt, 0), 1)
                lt = 1 - jnp.minimum(jnp.maximum(tt - (start[ex] + tot[ex]) + 1, 0), 1)
                inside = ge * lt
                acc = acc + inside * (ex - acc)
            etv[pl.ds(vi * _L, _L)] = acc
        pltpu.sync_copy(etv, et)


_HC = 1024
_NH = _H // _HC


def _mlp_body(et_ref, xs_ref, w1_ref, b1_ref, w2_ref, b2_ref, o_ref):
    e = et_ref[pl.program_id(0)]
    h = pl.program_id(1)

    @pl.when((e >= 0) & (h == 0))
    def _init():
        o_ref[...] = jnp.zeros_like(o_ref) + b2_ref[0]

    @pl.when(e >= 0)
    def _():
        act = (jnp.dot(xs_ref[...], w1_ref[0], preferred_element_type=jnp.float32)
               + b1_ref[0])
        act = jnp.maximum(act, 0.0)
        o_ref[...] += jnp.dot(act, w2_ref[0], preferred_element_type=jnp.float32)


def _mlp(et, xs, W1, b1, W2, b2):
    spec = pltpu.PrefetchScalarGridSpec(
        num_scalar_prefetch=1,
        grid=(_NT, _NH),
        in_specs=[
            pl.BlockSpec((_TM, _D), lambda m, h, et: (m, 0)),
            pl.BlockSpec((1, _D, _HC),
                         lambda m, h, et: (jnp.maximum(et[m], 0), 0, h)),
            pl.BlockSpec((1, 1, _HC),
                         lambda m, h, et: (jnp.maximum(et[m], 0), 0, h)),
            pl.BlockSpec((1, _HC, _D),
                         lambda m, h, et: (jnp.maximum(et[m], 0), h, 0)),
            pl.BlockSpec((1, 1, _D),
                         lambda m, h, et: (jnp.maximum(et[m], 0), 0, 0)),
        ],
        out_specs=pl.BlockSpec((_TM, _D), lambda m, h, et: (m, 0)),
    )
    return pl.pallas_call(
        _mlp_body, grid_spec=spec,
        out_shape=jax.ShapeDtypeStruct((_CAP, _D), jnp.float32),
    )(et, xs, W1, b1[:, None, :], W2, b2[:, None, :])


@functools.cache
def _combine_kernel():
    return pl.kernel(
        _combine_body,
        out_type=jax.ShapeDtypeStruct((_B, _D), jnp.float32),
        mesh=_sc_mesh(),
        compiler_params=pltpu.CompilerParams(needs_layout_passes=False),
        scratch_types=[
            pltpu.VMEM((_TOKW,), jnp.int32),
            pltpu.VMEM((_TOKW,), jnp.int32),
            pltpu.VMEM((_TOKW,), jnp.float32),
            pltpu.VMEM((_TOKW,), jnp.float32),
            pltpu.VMEM((_TOKW,), jnp.float32),
            pltpu.VMEM((16, _D), jnp.float32),
            pltpu.VMEM((16, _D), jnp.float32),
            pltpu.VMEM((16, _D), jnp.float32),
            pltpu.VMEM((16, _D), jnp.float32),
            pltpu.SemaphoreType.DMA,
            pltpu.SemaphoreType.DMA,
        ],
    )


def _combine_body(out2, x, g0, g1, gid, pos, out, p0v, p1v, g0v, g1v, gidv,
                  r0, r1, xbuf, obuf, sem0, sem1):
    w = lax.axis_index("c") * _NS + lax.axis_index("s")
    b0 = w * _TOKW
    pltpu.sync_copy(pos.at[pl.ds(b0, _TOKW)], p0v)
    pltpu.sync_copy(pos.at[pl.ds(_B + b0, _TOKW)], p1v)
    pltpu.sync_copy(g0.at[pl.ds(b0, _TOKW)], g0v)
    pltpu.sync_copy(g1.at[pl.ds(b0, _TOKW)], g1v)
    pltpu.sync_copy(gid.at[pl.ds(b0, _TOKW)], gidv)
    eps = jnp.float32(_EPS)
    for s in range(_TOKW // 16):
        c0 = pltpu.async_copy(out2.at[p0v.at[pl.ds(s * 16, 16)]], r0, sem0)
        c1 = pltpu.async_copy(out2.at[p1v.at[pl.ds(s * 16, 16)]], r1, sem1)
        pltpu.sync_copy(x.at[pl.ds(b0 + s * 16, 16)], xbuf)
        c0.wait()
        c1.wait()
        ga = g0v[pl.ds(s * 16, 16)]
        gb = g1v[pl.ds(s * 16, 16)]
        gc = gidv[pl.ds(s * 16, 16)]
        for t in range(16):
            a0, a1, a2 = ga[t], gb[t], gc[t]

            def inner(v, _, a0=a0, a1=a1, a2=a2, t=t):
                val = (a0 * r0[t, pl.ds(v * _L, _L)]
                       + a1 * r1[t, pl.ds(v * _L, _L)]
                       + a2 * xbuf[t, pl.ds(v * _L, _L)])
                sgn = jnp.sign(val)
                obuf[t, pl.ds(v * _L, _L)] = val + (1.0 - sgn * sgn) * eps
                return 0

            lax.fori_loop(0, _D // _L, inner, 0)
        pltpu.sync_copy(obuf, out.at[pl.ds(b0 + s * 16, 16)])


def kernel(x, w_gate, W1, b1, W2, b2):
    eids, gm, gid = _gate(x, w_gate)
    eflat = jnp.transpose(eids).reshape(2 * _B)
    xs, pos, et = _dispatch_kernel()(eflat, x)
    out2 = _mlp(et, xs, W1, b1, W2, b2)
    return _combine_kernel()(out2, x, gm[:, 0], gm[:, 1], gid[:, 0], pos)


# speedup vs baseline: 1.6762x; 1.6762x over previous
import functools

import numpy as np
import jax
import jax.numpy as jnp
from jax import lax
from jax.experimental import pallas as pl
from jax.experimental.pallas import tpu as pltpu
from jax.experimental.pallas import tpu_sc as plsc

_EPS = float(np.finfo(np.float64).eps)

_B = 2048
_D = 1024
_H = 4096
_E = 8
_NE = 7
_TM = 128
_CAP = 4992
_NT = _CAP // _TM
_XROWS = _CAP + _TM
_NC, _NS, _L = 2, 16, 16
_NW = _NC * _NS
_CHUNK = (2 * _B) // _NW
_TOKW = _B // _NW
_TN1 = 512
_TN2 = 512

@functools.cache
def _sc_mesh():
    return plsc.VectorSubcoreMesh(core_axis_name="c", subcore_axis_name="s",
                                  num_cores=_NC, num_subcores=_NS)


def _gate_body(x_ref, wg_ref, eids_ref, gm_ref, gid_ref):
    xf = x_ref[...]
    logits = jnp.dot(xf, wg_ref[...], preferred_element_type=jnp.float32)
    eidx = lax.broadcasted_iota(jnp.int32, logits.shape, 1)
    l1 = jnp.max(logits, axis=-1, keepdims=True)
    i1 = jnp.min(jnp.where(logits == l1, eidx, _E), axis=-1, keepdims=True)
    m1 = eidx == i1
    logits2 = jnp.where(m1, -jnp.inf, logits)
    l2 = jnp.max(logits2, axis=-1, keepdims=True)
    i2 = jnp.min(jnp.where(logits2 == l2, eidx, _E), axis=-1, keepdims=True)
    t = jnp.exp(l2 - l1)
    g1 = 1.0 / (1.0 + t)
    g2 = t / (1.0 + t)
    eids_ref[...] = jnp.concatenate([i1, i2], axis=1)
    gm_ref[...] = jnp.concatenate(
        [jnp.where(i1 == _NE, 0.0, g1), jnp.where(i2 == _NE, 0.0, g2)], axis=1)
    gid_ref[...] = jnp.where(i1 == _NE, g1, 0.0) + jnp.where(i2 == _NE, g2, 0.0)


def _gate(x, w_gate):
    return pl.pallas_call(
        _gate_body,
        out_shape=(
            jax.ShapeDtypeStruct((_B, 2), jnp.int32),
            jax.ShapeDtypeStruct((_B, 2), jnp.float32),
            jax.ShapeDtypeStruct((_B, 1), jnp.float32),
        ),
    )(x, w_gate)


@functools.cache
def _dispatch_kernel():
    return pl.kernel(
        _dispatch_body,
        out_type=(
            jax.ShapeDtypeStruct((_XROWS, _D), jnp.float32),
            jax.ShapeDtypeStruct((2 * _B,), jnp.int32),
            jax.ShapeDtypeStruct((48,), jnp.int32),
        ),
        mesh=_sc_mesh(),
        compiler_params=pltpu.CompilerParams(needs_layout_passes=False),
        scratch_types=[
            pltpu.VMEM((2 * _B,), jnp.int32),
            pltpu.VMEM((_CHUNK,), jnp.int32),
            pltpu.VMEM((64,), jnp.int32),
            pltpu.VMEM((64,), jnp.int32),
            pltpu.VMEM((64, _D), jnp.float32),
            pltpu.VMEM((48,), jnp.int32),
            pltpu.SemaphoreType.DMA,
        ],
    )


def _dispatch_body(eflat, x, xs, pos, et, ev, posv, ida, idb, xrow, etv, sem):
    w = lax.axis_index("c") * _NS + lax.axis_index("s")
    pltpu.sync_copy(eflat, ev)
    lanes = lax.iota(jnp.int32, _L)

    def scan_body(i, carry):
        ev_v = ev[pl.ds(i * _L, _L)]
        take = (i // (_CHUNK // _L) < w).astype(jnp.int32)
        out = []
        for ex in range(_NE):
            eq = 1 - jnp.minimum(jnp.abs(ev_v - ex), 1)
            out.append(carry[ex] + eq)
            out.append(carry[_NE + ex] + eq * take)
        return tuple(out[0::2]) + tuple(out[1::2])

    zero = jnp.zeros((_L,), jnp.int32)
    acc = lax.fori_loop(0, (2 * _B) // _L, scan_body, (zero,) * (2 * _NE))

    tot = jnp.zeros((_L,), jnp.int32)
    base = jnp.zeros((_L,), jnp.int32)
    for ex in range(_NE):
        sel = 1 - jnp.minimum(jnp.abs(lanes - ex), 1)
        tot = tot + sel * plsc.cumsum(acc[ex])[_L - 1]
        base = base + sel * plsc.cumsum(acc[_NE + ex])[_L - 1]

    is_mlp_lane = 1 - jnp.minimum(jnp.maximum(lanes - (_NE - 1), 0), 1)
    padded = is_mlp_lane * (((tot + (_TM - 1)) >> 7) << 7)
    cum = plsc.cumsum(padded)
    start = cum - padded
    mybase = start + base

    pbase = w * _CHUNK
    runs = [jnp.int32(0)] * _NE
    for v in range(_CHUNK // _L):
        ev_v = ev[pl.ds(pbase + v * _L, _L)]
        slot = jnp.zeros((_L,), jnp.int32)
        for ex in range(_NE):
            eq = 1 - jnp.minimum(jnp.abs(ev_v - ex), 1)
            rank = plsc.cumsum(eq)
            bs = mybase[ex] + runs[ex]
            slot = slot + eq * (bs + rank - 1)
            runs[ex] = runs[ex] + rank[_L - 1]
        is_mlp = 1 - jnp.minimum(jnp.maximum(ev_v - (_NE - 1), 0), 1)
        posv[pl.ds(v * _L, _L)] = is_mlp * slot
        sl_sc = is_mlp * slot + (1 - is_mlp) * _CAP
        if v < 4:
            ida[pl.ds(v * _L, _L)] = sl_sc
        else:
            idb[pl.ds((v - 4) * _L, _L)] = sl_sc

    pltpu.sync_copy(posv, pos.at[pl.ds(pbase, _CHUNK)])

    brow = (w % _NS) * _CHUNK
    for half in range(2):
        pltpu.sync_copy(x.at[pl.ds(brow + half * 64, 64)], xrow)
        idx = ida if half == 0 else idb
        pltpu.async_copy(xrow, xs.at[idx], sem).wait()

    @pl.when(w == 0)
    def _et():
        for vi in range(48 // _L):
            tt = (lax.iota(jnp.int32, _L) + vi * _L) * _TM
            acc = jnp.full((_L,), -1, jnp.int32)
            for ex in range(_NE):
                ge = 1 - jnp.minimum(jnp.maximum(start[ex] - tt, 0), 1)
                lt = 1 - jnp.minimum(jnp.maximum(tt - (start[ex] + tot[ex]) + 1, 0), 1)
                inside = ge * lt
                acc = acc + inside * (ex - acc)
            etv[pl.ds(vi * _L, _L)] = acc
        pltpu.sync_copy(etv, et)


def _l1_body(et_ref, xs_ref, w1_ref, b1_ref, h1_ref):
    e = et_ref[pl.program_id(0)]

    @pl.when(e >= 0)
    def _():
        acc = (jnp.dot(xs_ref[...], w1_ref[0], preferred_element_type=jnp.float32)
               + b1_ref[0])
        h1_ref[...] = jnp.maximum(acc, 0.0).astype(jnp.bfloat16)


def _l1(et, xs, W1, b1):
    spec = pltpu.PrefetchScalarGridSpec(
        num_scalar_prefetch=1,
        grid=(_NT,),
        in_specs=[
            pl.BlockSpec((_TM, _D), lambda m, et: (m, 0)),
            pl.BlockSpec((1, _D, _H), lambda m, et: (jnp.where(et[m] < 0, _NE - 1, et[m]), 0, 0)),
            pl.BlockSpec((1, 1, _H), lambda m, et: (jnp.where(et[m] < 0, _NE - 1, et[m]), 0, 0)),
        ],
        out_specs=pl.BlockSpec((_TM, _H), lambda m, et: (m, 0)),
    )
    return pl.pallas_call(
        _l1_body, grid_spec=spec,
        out_shape=jax.ShapeDtypeStruct((_CAP, _H), jnp.bfloat16),
    )(et, xs, W1, b1[:, None, :])


def _l2_body(et_ref, h1_ref, w2_ref, b2_ref, o_ref):
    e = et_ref[pl.program_id(0)]

    @pl.when(e >= 0)
    def _():
        o_ref[...] = (jnp.dot(h1_ref[...], w2_ref[0],
                              preferred_element_type=jnp.float32)
                      + b2_ref[0])


def _l2(et, h1, W2, b2):
    spec = pltpu.PrefetchScalarGridSpec(
        num_scalar_prefetch=1,
        grid=(_NT,),
        in_specs=[
            pl.BlockSpec((_TM, _H), lambda m, et: (m, 0)),
            pl.BlockSpec((1, _H, _D), lambda m, et: (jnp.where(et[m] < 0, _NE - 1, et[m]), 0, 0)),
            pl.BlockSpec((1, 1, _D), lambda m, et: (jnp.where(et[m] < 0, _NE - 1, et[m]), 0, 0)),
        ],
        out_specs=pl.BlockSpec((_TM, _D), lambda m, et: (m, 0)),
    )
    return pl.pallas_call(
        _l2_body, grid_spec=spec,
        out_shape=jax.ShapeDtypeStruct((_CAP, _D), jnp.float32),
    )(et, h1, W2, b2[:, None, :])


@functools.cache
def _combine_kernel():
    return pl.kernel(
        _combine_body,
        out_type=jax.ShapeDtypeStruct((_B, _D), jnp.float32),
        mesh=_sc_mesh(),
        compiler_params=pltpu.CompilerParams(needs_layout_passes=False),
        scratch_types=[
            pltpu.VMEM((_TOKW,), jnp.int32),
            pltpu.VMEM((_TOKW,), jnp.int32),
            pltpu.VMEM((_TOKW,), jnp.float32),
            pltpu.VMEM((_TOKW,), jnp.float32),
            pltpu.VMEM((_TOKW,), jnp.float32),
            [pltpu.VMEM((16, _D), jnp.float32)] * 2,
            [pltpu.VMEM((16, _D), jnp.float32)] * 2,
            [pltpu.VMEM((16, _D), jnp.float32)] * 2,
            pltpu.VMEM((16, _D), jnp.float32),
            [pltpu.SemaphoreType.DMA] * 2,
            [pltpu.SemaphoreType.DMA] * 2,
            [pltpu.SemaphoreType.DMA] * 2,
        ],
    )


def _combine_body(out2, x, g0, g1, gid, pos, out, p0v, p1v, g0v, g1v, gidv,
                  r0, r1, xb, obuf, sem0, sem1, semx):
    w = lax.axis_index("c") * _NS + lax.axis_index("s")
    b0 = w * _TOKW
    pltpu.sync_copy(pos.at[pl.ds(b0, _TOKW)], p0v)
    pltpu.sync_copy(pos.at[pl.ds(_B + b0, _TOKW)], p1v)
    pltpu.sync_copy(g0.at[pl.ds(b0, _TOKW)], g0v)
    pltpu.sync_copy(g1.at[pl.ds(b0, _TOKW)], g1v)
    pltpu.sync_copy(gid.at[pl.ds(b0, _TOKW)], gidv)
    eps = jnp.float32(_EPS)
    nchunk = _TOKW // 16

    def fire(sc):
        k = sc % 2
        return (
            pltpu.async_copy(out2.at[p0v.at[pl.ds(sc * 16, 16)]], r0[k], sem0[k]),
            pltpu.async_copy(out2.at[p1v.at[pl.ds(sc * 16, 16)]], r1[k], sem1[k]),
            pltpu.async_copy(x.at[pl.ds(b0 + sc * 16, 16)], xb[k], semx[k]),
        )

    pend = fire(0)
    for sc in range(nchunk):
        k = sc % 2
        for c in pend:
            c.wait()
        if sc + 1 < nchunk:
            pend = fire(sc + 1)
        ga = g0v[pl.ds(sc * 16, 16)]
        gb = g1v[pl.ds(sc * 16, 16)]
        gc = gidv[pl.ds(sc * 16, 16)]
        for t in range(16):
            a0, a1, a2 = ga[t], gb[t], gc[t]

            def inner(v, _, a0=a0, a1=a1, a2=a2, t=t, k=k):
                for u in range(4):
                    sl = pl.ds((v * 4 + u) * _L, _L)
                    val = (a0 * r0[k][t, sl] + a1 * r1[k][t, sl]
                           + a2 * xb[k][t, sl])
                    sgn = jnp.sign(val)
                    obuf[t, sl] = val + (1.0 - sgn * sgn) * eps
                return 0

            lax.fori_loop(0, _D // (_L * 4), inner, 0)
        pltpu.sync_copy(obuf, out.at[pl.ds(b0 + sc * 16, 16)])


def kernel(x, w_gate, W1, b1, W2, b2):
    eids, gm, gid = _gate(x, w_gate)
    eflat = jnp.transpose(eids).reshape(2 * _B)
    xs, pos, et = _dispatch_kernel()(eflat, x)
    h1 = _l1(et, xs, W1, b1)
    out2 = _l2(et, h1, W2, b2)
    return _combine_kernel()(out2, x, gm[:, 0], gm[:, 1], gid[:, 0], pos)
